# baseline (speedup 1.0000x reference)
import jax
import jax.numpy as jnp
from jax.experimental import pallas as pl
from jax.experimental.pallas import tpu as pltpu

_N = 10000
_BM = 400
_NS = _N // _BM


def _pass1_kernel(x_ref, w1_ref, b1_ref, w2_ref, l_ref,
                  s2_ref, lbf_ref, s1_ref):
    i = pl.program_id(0)

    @pl.when(i == 0)
    def _():
        s1_ref[...] = jnp.dot(x_ref[...], w1_ref[...],
                              preferred_element_type=jnp.float32)

    @pl.when(i >= 1)
    def _():
        stripe = l_ref[...]
        lbf_ref[...] = stripe.astype(jnp.bfloat16)
        h = jnp.dot(stripe, s1_ref[...],
                    preferred_element_type=jnp.float32)
        h = jnp.maximum(h + b1_ref[...], 0.0)
        s2_ref[...] = jnp.dot(h, w2_ref[...],
                              preferred_element_type=jnp.float32)


def _pass2_kernel(lbf_ref, s2_ref, b2_ref, o_ref):
    o_ref[...] = jnp.dot(lbf_ref[...], s2_ref[...],
                         preferred_element_type=jnp.float32) + b2_ref[...]


def _l_stripe(i):
    return (jnp.maximum(i - 1, 0), 0)


def kernel(Laplacian, feature, W1, b1, W2, b2):
    n, in_dim = feature.shape
    n_hid = W1.shape[1]
    out_dim = W2.shape[1]
    b1r = b1.reshape(1, n_hid)
    b2r = b2.reshape(1, out_dim)

    s2, lbf = pl.pallas_call(
        _pass1_kernel,
        grid=(1 + _NS,),
        in_specs=[
            pl.BlockSpec((n, in_dim), lambda i: (0, 0)),
            pl.BlockSpec((in_dim, n_hid), lambda i: (0, 0)),
            pl.BlockSpec((1, n_hid), lambda i: (0, 0)),
            pl.BlockSpec((n_hid, out_dim), lambda i: (0, 0)),
            pl.BlockSpec((_BM, n), _l_stripe),
        ],
        out_specs=[
            pl.BlockSpec((_BM, out_dim), _l_stripe),
            pl.BlockSpec((_BM, n), _l_stripe),
        ],
        out_shape=[
            jax.ShapeDtypeStruct((n, out_dim), jnp.float32),
            jax.ShapeDtypeStruct((n, n), jnp.bfloat16),
        ],
        scratch_shapes=[pltpu.VMEM((n, n_hid), jnp.float32)],
        compiler_params=pltpu.CompilerParams(
            dimension_semantics=("arbitrary",)),
    )(feature, W1, b1r, W2, Laplacian)

    return pl.pallas_call(
        _pass2_kernel,
        grid=(_NS,),
        in_specs=[
            pl.BlockSpec((_BM, n), lambda i: (i, 0)),
            pl.BlockSpec((n, out_dim), lambda i: (0, 0)),
            pl.BlockSpec((1, out_dim), lambda i: (0, 0)),
        ],
        out_specs=pl.BlockSpec((_BM, out_dim), lambda i: (i, 0)),
        out_shape=jax.ShapeDtypeStruct((n, out_dim), jnp.float32),
        compiler_params=pltpu.CompilerParams(
            dimension_semantics=("arbitrary",)),
    )(lbf, s2, b2r)


# speedup vs baseline: 1.0584x; 1.0584x over previous
import jax
import jax.numpy as jnp
from jax.experimental import pallas as pl
from jax.experimental.pallas import tpu as pltpu

_N = 10000
_BM = 400
_NS = _N // _BM
_R = 3


def _stripe_idx(t):
    return jnp.where(t < _NS, t, t - _NS)


def _s1_kernel(x_ref, w1_ref, s1_ref):
    s1_ref[...] = jnp.dot(x_ref[...], w1_ref[...],
                          preferred_element_type=jnp.float32)


def _fused_kernel(s1_ref, b1_ref, w2_ref, b2_ref, l_ref,
                  o_ref, ring0, ring1, ring2, s2_ref, sems):
    rings = (ring0, ring1, ring2)

    def start_fetch(t, k):
        pltpu.make_async_copy(
            l_ref.at[pl.ds(_stripe_idx(t) * _BM, _BM), :],
            rings[k],
            sems.at[k],
        ).start()

    def wait_fetch(t, k):
        pltpu.make_async_copy(
            l_ref.at[pl.ds(_stripe_idx(t) * _BM, _BM), :],
            rings[k],
            sems.at[k],
        ).wait()

    for t in range(_R):
        start_fetch(t, t)

    def step(t, k):
        wait_fetch(t, k)

        @pl.when(t < _NS)
        def _():
            h = jnp.dot(rings[k][...], s1_ref[...],
                        preferred_element_type=jnp.float32)
            h = jnp.maximum(h + b1_ref[...], 0.0)
            s2_ref[pl.ds(t * _BM, _BM), :] = jnp.dot(
                h, w2_ref[...], preferred_element_type=jnp.float32)

        @pl.when(t >= _NS)
        def _():
            o_ref[pl.ds((t - _NS) * _BM, _BM), :] = (
                jnp.dot(rings[k][...], s2_ref[...],
                        preferred_element_type=jnp.float32) + b2_ref[...])

        @pl.when(t + _R < 2 * _NS)
        def _():
            start_fetch(t + _R, k)

    def body(t, _):
        slot = jax.lax.rem(t, _R)
        for k in range(_R):
            @pl.when(slot == k)
            def _(k=k):
                step(t, k)
        return 0

    jax.lax.fori_loop(0, 2 * _NS, body, 0)


def kernel(Laplacian, feature, W1, b1, W2, b2):
    n, in_dim = feature.shape
    n_hid = W1.shape[1]
    out_dim = W2.shape[1]
    b1r = b1.reshape(1, n_hid)
    b2r = b2.reshape(1, out_dim)

    s1 = pl.pallas_call(
        _s1_kernel,
        in_specs=[
            pl.BlockSpec((n, in_dim), lambda: (0, 0)),
            pl.BlockSpec((in_dim, n_hid), lambda: (0, 0)),
        ],
        out_specs=pl.BlockSpec((n, n_hid), lambda: (0, 0)),
        out_shape=jax.ShapeDtypeStruct((n, n_hid), jnp.float32),
    )(feature, W1)

    return pl.pallas_call(
        _fused_kernel,
        in_specs=[
            pl.BlockSpec((n, n_hid), lambda: (0, 0)),
            pl.BlockSpec((1, n_hid), lambda: (0, 0)),
            pl.BlockSpec((n_hid, out_dim), lambda: (0, 0)),
            pl.BlockSpec((1, out_dim), lambda: (0, 0)),
            pl.BlockSpec(memory_space=pl.ANY),
        ],
        out_specs=pl.BlockSpec((n, out_dim), lambda: (0, 0)),
        out_shape=jax.ShapeDtypeStruct((n, out_dim), jnp.float32),
        scratch_shapes=[
            pltpu.VMEM((_BM, n), jnp.float32),
            pltpu.VMEM((_BM, n), jnp.float32),
            pltpu.VMEM((_BM, n), jnp.float32),
            pltpu.VMEM((n, out_dim), jnp.float32),
            pltpu.SemaphoreType.DMA((_R,)),
        ],
        compiler_params=pltpu.CompilerParams(
            vmem_limit_bytes=67108864),
    )(s1, b1r, W2, b2r, Laplacian)
